# Initial kernel scaffold; baseline (speedup 1.0000x reference)
import jax
import jax.numpy as jnp
from jax.experimental import pallas as pl


def kernel(x, adj, W_conv, b_conv, prelu_a, W_lin, b_lin):
    raise NotImplementedError("write your pallas kernel here")


# speedup vs baseline: 21.6834x; 21.6834x over previous
import functools

import jax
import jax.numpy as jnp
from jax import lax
from jax.experimental import pallas as pl
from jax.experimental.pallas import tpu as pltpu
from jax.experimental.pallas import tpu_sc as plsc

NC = 2
NS = 16
NW = NC * NS

N = 10000
NP = 10240
NPR = NP // 128
E = 320000
EPT = E // NW
CHUNK = 80
NITER = EPT // CHUNK
RPT = NP // NS


def _mesh():
    return plsc.VectorSubcoreMesh(
        core_axis_name="c", subcore_axis_name="s", num_cores=NC, num_subcores=NS
    )


def _sc_hist(col):
    @functools.partial(
        pl.kernel,
        out_type=jax.ShapeDtypeStruct((NW, NP), jnp.float32),
        mesh=_mesh(),
        compiler_params=pltpu.CompilerParams(needs_layout_passes=False),
        scratch_types=[
            pltpu.VMEM((EPT,), jnp.int32),
            pltpu.VMEM((NP,), jnp.float32),
        ],
    )
    def k(col_hbm, out_hbm, col_v, hist_v):
        cid = lax.axis_index("c")
        sid = lax.axis_index("s")
        wid = cid * NS + sid

        def zbody(i, carry):
            hist_v[pl.ds(i * 16, 16)] = jnp.zeros((16,), jnp.float32)
            return carry

        lax.fori_loop(0, NP // 16, zbody, 0)
        pltpu.sync_copy(col_hbm.at[pl.ds(wid * EPT, EPT)], col_v)
        ones = jnp.ones((16,), jnp.float32)

        def hbody(i, carry):
            idx = col_v[pl.ds(i * 16, 16)]
            plsc.addupdate_scatter(hist_v, [idx], ones)
            return carry

        lax.fori_loop(0, EPT // 16, hbody, 0)
        pltpu.sync_copy(hist_v, out_hbm.at[wid])

    return k(col)


def _sc_scatter(row3d, col3d, g):
    @functools.partial(
        pl.kernel,
        out_type=jax.ShapeDtypeStruct((NC, NP, 128), jnp.float32),
        mesh=_mesh(),
        compiler_params=pltpu.CompilerParams(needs_layout_passes=False),
        scratch_types=[
            pltpu.VMEM((NITER, CHUNK), jnp.int32),
            pltpu.VMEM((NITER, CHUNK), jnp.int32),
            pltpu.VMEM((CHUNK, 128), jnp.float32),
            pltpu.VMEM_SHARED((NP, 128), jnp.float32),
            pltpu.SemaphoreType.DMA,
        ],
    )
    def k(row_hbm, col_hbm, g_hbm, out_hbm, ridx, cidx, rows_v, acc, sem):
        cid = lax.axis_index("c")
        sid = lax.axis_index("s")
        wid = cid * NS + sid

        def zbody(i, carry):
            r = i // 8
            cc = (i % 8) * 16
            rows_v[r, pl.ds(cc, 16)] = jnp.zeros((16,), jnp.float32)
            return carry

        lax.fori_loop(0, CHUNK * 8, zbody, 0)
        for b in range(RPT // CHUNK):
            pltpu.sync_copy(rows_v, acc.at[pl.ds(sid * RPT + b * CHUNK, CHUNK), :])
        pltpu.sync_copy(row_hbm.at[wid], ridx)
        pltpu.sync_copy(col_hbm.at[wid], cidx)
        plsc.subcore_barrier()

        def body(i, carry):
            pltpu.async_copy(g_hbm.at[ridx.at[i]], rows_v, sem).wait()
            pltpu.sync_copy(rows_v, acc.at[cidx.at[i]], add=True)
            return carry

        lax.fori_loop(0, NITER, body, 0)
        plsc.subcore_barrier()
        pltpu.sync_copy(
            acc.at[pl.ds(sid * RPT, RPT), :],
            out_hbm.at[cid, pl.ds(sid * RPT, RPT), :],
        )

    return k(row3d, col3d, g)


def _diag(dinv):
    ii = lax.broadcasted_iota(jnp.int32, (128, 128), 0)
    jj = lax.broadcasted_iota(jnp.int32, (128, 128), 1)
    return jnp.where(ii == jj, jnp.broadcast_to(dinv, (128, 128)), 0.0)


def _tc_g(parts, xp, w):
    def body(p_ref, x_ref, w_ref, o_ref):
        d = jnp.sum(p_ref[0], axis=0, keepdims=True) + 1.0
        dinv = lax.rsqrt(d)
        h = jnp.dot(x_ref[...], w_ref[...], preferred_element_type=jnp.float32)
        o_ref[...] = jnp.dot(_diag(dinv), h, preferred_element_type=jnp.float32)

    return pl.pallas_call(
        body,
        grid=(NPR,),
        in_specs=[
            pl.BlockSpec((1, NW, 128), lambda i: (i, 0, 0)),
            pl.BlockSpec((128, 128), lambda i: (i, 0)),
            pl.BlockSpec((128, 128), lambda i: (0, 0)),
        ],
        out_specs=pl.BlockSpec((128, 128), lambda i: (i, 0)),
        out_shape=jax.ShapeDtypeStruct((NP, 128), jnp.float32),
    )(parts, xp, w)


def _tc_out(sparts, g, degparts, bc, pa, wl, bl):
    def body(s_ref, g_ref, p_ref, bc_ref, pa_ref, wl_ref, bl_ref, o_ref):
        d = jnp.sum(p_ref[0], axis=0, keepdims=True) + 1.0
        dinv = lax.rsqrt(d)
        s = s_ref[0] + s_ref[1] + g_ref[...]
        agg = jnp.dot(_diag(dinv), s, preferred_element_type=jnp.float32)
        agg = agg + bc_ref[...]
        a = pa_ref[0, 0]
        act = jnp.where(agg > 0, agg, a * agg)
        o_ref[...] = (
            jnp.dot(act, wl_ref[...], preferred_element_type=jnp.float32) + bl_ref[...]
        )

    return pl.pallas_call(
        body,
        grid=(NPR,),
        in_specs=[
            pl.BlockSpec((NC, 128, 128), lambda i: (0, i, 0)),
            pl.BlockSpec((128, 128), lambda i: (i, 0)),
            pl.BlockSpec((1, NW, 128), lambda i: (i, 0, 0)),
            pl.BlockSpec((1, 128), lambda i: (0, 0)),
            pl.BlockSpec((1, 1), lambda i: (0, 0)),
            pl.BlockSpec((128, 128), lambda i: (0, 0)),
            pl.BlockSpec((1, 128), lambda i: (0, 0)),
        ],
        out_specs=pl.BlockSpec((128, 128), lambda i: (i, 0)),
        out_shape=jax.ShapeDtypeStruct((NP, 128), jnp.float32),
    )(sparts, g, degparts, bc, pa, wl, bl)


def kernel(x, adj, W_conv, b_conv, prelu_a, W_lin, b_lin):
    n = x.shape[0]
    row = adj[0]
    col = adj[1]
    degparts = _sc_hist(col).reshape(NW, NPR, 128).transpose(1, 0, 2)
    xp = jnp.pad(x, ((0, NP - n), (0, 0)))
    g = _tc_g(degparts, xp, W_conv)
    sparts = _sc_scatter(
        row.reshape(NW, NITER, CHUNK), col.reshape(NW, NITER, CHUNK), g
    )
    outp = _tc_out(
        sparts,
        g,
        degparts,
        b_conv.reshape(1, 128),
        jnp.asarray(prelu_a, jnp.float32).reshape(1, 1),
        W_lin,
        b_lin.reshape(1, 128),
    )
    return outp[:n]
